# baseline (speedup 1.0000x reference)
import functools

import jax
import jax.numpy as jnp
from jax import lax
from jax.experimental import pallas as pl
from jax.experimental.pallas import tpu as pltpu
from jax.experimental.pallas import tpu_sc as plsc

VOCAB = 1000000
D = 128
ODIM = 5
B = 4096
H = 200
LANES = 16
DCH = D // LANES

CH0, CH1 = 104, 96
NBUF = 3


def _sc_info():
    try:
        info = plsc.get_sparse_core_info()
        return info.num_cores, info.num_subcores
    except Exception:
        return 2, 16


def _make_cbow_kernel():
    nc, ns = _sc_info()
    nw = nc * ns
    b_per_w = B // nw
    mesh = plsc.VectorSubcoreMesh(
        core_axis_name="c", subcore_axis_name="s",
        num_cores=nc, num_subcores=ns)

    @functools.partial(
        pl.kernel,
        out_type=jax.ShapeDtypeStruct((B, LANES), jnp.float32),
        mesh=mesh,
        scratch_types=[
            pltpu.VMEM((b_per_w * H,), jnp.int32),
            pltpu.VMEM((NBUF, H, D), jnp.float32),
            pltpu.VMEM((b_per_w, LANES), jnp.float32),
            pltpu.VMEM((ODIM, D), jnp.float32),
            pltpu.VMEM((LANES,), jnp.float32),
            pltpu.SemaphoreType.DMA,
            pltpu.SemaphoreType.DMA,
            pltpu.SemaphoreType.DMA,
            pltpu.SemaphoreType.DMA,
            pltpu.SemaphoreType.DMA,
            pltpu.SemaphoreType.DMA,
        ],
    )
    def cbow_kernel(idx_hbm, table_hbm, w_hbm, bvec_hbm, out_hbm,
                    idx_v, rows_v, stage_v, w_v, bvec_v,
                    sem0a, sem0b, sem1a, sem1b, sem2a, sem2b):
        cid = lax.axis_index("c")
        sid = lax.axis_index("s")
        wid = sid * nc + cid
        base = wid * b_per_w

        pltpu.sync_copy(idx_hbm.at[pl.ds(base * H, b_per_w * H)], idx_v)
        pltpu.sync_copy(w_hbm, w_v)
        pltpu.sync_copy(bvec_hbm, bvec_v)

        sems = ((sem0a, sem0b), (sem1a, sem1b), (sem2a, sem2b))

        def gather_copies(b, buf):
            off = b * H
            c0 = pltpu.make_async_copy(
                table_hbm.at[idx_v.at[pl.ds(off, CH0)]],
                rows_v.at[buf, pl.ds(0, CH0)],
                sems[buf][0],
            )
            c1 = pltpu.make_async_copy(
                table_hbm.at[idx_v.at[pl.ds(off + CH0, CH1)]],
                rows_v.at[buf, pl.ds(CH0, CH1)],
                sems[buf][1],
            )
            return c0, c1

        def issue(b, buf):
            c0, c1 = gather_copies(b, buf)
            c0.start()
            c1.start()

        for b0 in range(NBUF):
            issue(b0, b0)

        bvec = bvec_v[...]
        lane_ids = lax.iota(jnp.int32, LANES)
        masks = tuple(lane_ids == o for o in range(ODIM))
        perms = tuple(
            lax.rem(lane_ids + sh, jnp.int32(LANES)) for sh in (8, 4, 2, 1))

        gather_dnums = lax.GatherDimensionNumbers(
            offset_dims=(), collapsed_slice_dims=(0,), start_index_map=(0,))

        def lane_shuffle(m, p):
            return lax.gather(
                m, p[:, None], gather_dnums, slice_sizes=(1,),
                mode=lax.GatherScatterMode.PROMISE_IN_BOUNDS)

        def lane_total(m):
            for p in perms:
                m = m + lane_shuffle(m, p)
            return m

        def acc_range(buf, lo, hi, accs):
            def acc_body(l, accs):
                return tuple(
                    accs[j] + rows_v[buf, l, pl.ds(LANES * j, LANES)]
                    for j in range(DCH)
                )
            return lax.fori_loop(lo, hi, acc_body, accs)

        def process(b, buf):
            c0, c1 = gather_copies(b, buf)
            if isinstance(b, int):
                nb = min(b + NBUF, b_per_w - 1)
            else:
                nb = jnp.minimum(b + NBUF, b_per_w - 1)
            n0, n1 = gather_copies(nb, buf)
            accs = tuple(
                jnp.zeros((LANES,), jnp.float32) for _ in range(DCH))
            c0.wait()
            accs = acc_range(buf, 0, CH0, accs)

            @pl.when(b + NBUF < b_per_w)
            def _():
                n0.start()
            c1.wait()
            accs = acc_range(buf, CH0, H, accs)

            @pl.when(b + NBUF < b_per_w)
            def _():
                n1.start()

            out_vec = bvec
            for o in range(ODIM):
                m = accs[0] * w_v[o, pl.ds(0, LANES)]
                for j in range(1, DCH):
                    m = m + accs[j] * w_v[o, pl.ds(LANES * j, LANES)]
                out_vec = jnp.where(masks[o], lane_total(m), out_vec)
            stage_v[b, :] = out_vec

        def outer(i, carry):
            for buf in range(NBUF):
                b = i * NBUF + buf
                process(b, buf)
            return carry

        n_full = b_per_w // NBUF
        lax.fori_loop(0, n_full, outer, 0)
        for b in range(n_full * NBUF, b_per_w):
            buf = b % NBUF
            process(b, buf)

        pltpu.sync_copy(stage_v, out_hbm.at[pl.ds(base, b_per_w)])

    return cbow_kernel


def kernel(inputs, embed_weight, linear_w, linear_b):
    idx_flat = jnp.reshape(inputs, (B * H,)).astype(jnp.int32)
    bvec = jnp.pad(linear_b, (0, LANES - ODIM)) * jnp.float32(H)
    out16 = _make_cbow_kernel()(idx_flat, embed_weight, linear_w, bvec)
    return out16[:, :ODIM]


# speedup vs baseline: 1.0049x; 1.0049x over previous
import functools

import jax
import jax.numpy as jnp
from jax import lax
from jax.experimental import pallas as pl
from jax.experimental.pallas import tpu as pltpu
from jax.experimental.pallas import tpu_sc as plsc

VOCAB = 1000000
D = 128
ODIM = 5
B = 4096
H = 200
LANES = 16
DCH = D // LANES

CH0, CH1 = 128, 72
NBUF = 3


def _sc_info():
    try:
        info = plsc.get_sparse_core_info()
        return info.num_cores, info.num_subcores
    except Exception:
        return 2, 16


def _make_pooled_kernel():
    nc, ns = _sc_info()
    nw = nc * ns
    b_per_w = B // nw
    mesh = plsc.VectorSubcoreMesh(
        core_axis_name="c", subcore_axis_name="s",
        num_cores=nc, num_subcores=ns)

    @functools.partial(
        pl.kernel,
        out_type=jax.ShapeDtypeStruct((B, D), jnp.float32),
        mesh=mesh,
        scratch_types=[
            pltpu.VMEM((b_per_w * H,), jnp.int32),
            pltpu.VMEM((NBUF, H, D), jnp.float32),
            pltpu.VMEM((b_per_w, D), jnp.float32),
            pltpu.SemaphoreType.DMA,
            pltpu.SemaphoreType.DMA,
            pltpu.SemaphoreType.DMA,
            pltpu.SemaphoreType.DMA,
            pltpu.SemaphoreType.DMA,
            pltpu.SemaphoreType.DMA,
        ],
    )
    def pooled_kernel(idx_hbm, table_hbm, out_hbm, idx_v, rows_v, stage_v,
                      sem0a, sem0b, sem1a, sem1b, sem2a, sem2b):
        cid = lax.axis_index("c")
        sid = lax.axis_index("s")
        wid = sid * nc + cid
        base = wid * b_per_w

        pltpu.sync_copy(idx_hbm.at[pl.ds(base * H, b_per_w * H)], idx_v)

        sems = ((sem0a, sem0b), (sem1a, sem1b), (sem2a, sem2b))

        def gather_copies(b, buf):
            off = b * H
            c0 = pltpu.make_async_copy(
                table_hbm.at[idx_v.at[pl.ds(off, CH0)]],
                rows_v.at[buf, pl.ds(0, CH0)],
                sems[buf][0],
            )
            c1 = pltpu.make_async_copy(
                table_hbm.at[idx_v.at[pl.ds(off + CH0, CH1)]],
                rows_v.at[buf, pl.ds(CH0, CH1)],
                sems[buf][1],
            )
            return c0, c1

        def issue(b, buf):
            c0, c1 = gather_copies(b, buf)
            c0.start()
            c1.start()

        for b0 in range(NBUF):
            issue(b0, b0)

        def acc_range(buf, lo, hi, accs):
            def acc_body(l, accs):
                return tuple(
                    accs[j] + rows_v[buf, l, pl.ds(LANES * j, LANES)]
                    for j in range(DCH)
                )
            return lax.fori_loop(lo, hi, acc_body, accs)

        def process(b, buf):
            c0, c1 = gather_copies(b, buf)
            if isinstance(b, int):
                nb = min(b + NBUF, b_per_w - 1)
            else:
                nb = jnp.minimum(b + NBUF, b_per_w - 1)
            n0, n1 = gather_copies(nb, buf)
            accs = tuple(
                jnp.zeros((LANES,), jnp.float32) for _ in range(DCH))
            c0.wait()
            accs = acc_range(buf, 0, CH0, accs)

            @pl.when(b + NBUF < b_per_w)
            def _():
                n0.start()
            c1.wait()
            accs = acc_range(buf, CH0, H, accs)

            @pl.when(b + NBUF < b_per_w)
            def _():
                n1.start()
            for j in range(DCH):
                stage_v[b, pl.ds(LANES * j, LANES)] = accs[j]

        def outer(i, carry):
            for buf in range(NBUF):
                b = i * NBUF + buf
                process(b, buf)
            return carry

        n_full = b_per_w // NBUF
        lax.fori_loop(0, n_full, outer, 0)
        for b in range(n_full * NBUF, b_per_w):
            buf = b % NBUF
            process(b, buf)

        pltpu.sync_copy(stage_v, out_hbm.at[pl.ds(base, b_per_w)])

    return pooled_kernel


def _mm_body(p_ref, wt_ref, b_ref, o_ref):
    o_ref[...] = (
        jnp.dot(p_ref[...], wt_ref[...], preferred_element_type=jnp.float32)
        + jnp.float32(H) * b_ref[...]
    )


def kernel(inputs, embed_weight, linear_w, linear_b):
    idx_flat = jnp.reshape(inputs, (B * H,)).astype(jnp.int32)
    pooled = _make_pooled_kernel()(idx_flat, embed_weight)
    out = pl.pallas_call(
        _mm_body,
        out_shape=jax.ShapeDtypeStruct((B, ODIM), jnp.float32),
    )(pooled, linear_w.T, jnp.reshape(linear_b, (1, ODIM)))
    return out
